# baseline (speedup 1.0000x reference)
import functools

import jax
import jax.numpy as jnp
from jax import lax
from jax.experimental import pallas as pl
from jax.experimental.pallas import tpu as pltpu
from jax.experimental.pallas import tpu_sc as plsc

NC = 2
NS = 16
NW = NC * NS
CHUNK = 50

_HIGH = lax.Precision.HIGHEST


def _segment_sum_sc(ylaug, src, dst, zeros):
    _, dp = ylaug.shape
    n = zeros.shape[0]
    e = src.shape[0]
    epw = e // NW
    steps = epw // CHUNK
    rpt = n // NS
    mesh = plsc.VectorSubcoreMesh(
        core_axis_name="c", subcore_axis_name="s", num_cores=NC, num_subcores=NS)

    @functools.partial(
        pl.kernel,
        out_type=jax.ShapeDtypeStruct((NC, n, dp), jnp.float32),
        mesh=mesh,
        scratch_types=[
            pltpu.VMEM((steps, 2, CHUNK), jnp.int32),
            pltpu.VMEM((CHUNK, dp), jnp.float32),
            pltpu.VMEM((CHUNK, dp), jnp.float32),
            pltpu.VMEM_SHARED((n, dp), jnp.float32),
            pltpu.SemaphoreType.DMA,
            pltpu.SemaphoreType.DMA,
            pltpu.SemaphoreType.DMA,
            pltpu.SemaphoreType.DMA,
        ],
        compiler_params=pltpu.CompilerParams(use_tc_tiling_on_sc=False),
    )
    def segsum(yl_hbm, sd_hbm, z_hbm, out_hbm,
               idx_v, rows_a, rows_b, acc, sem_ga, sem_gb, sem_sa, sem_sb):
        c = lax.axis_index("c")
        s = lax.axis_index("s")
        wid = s * NC + c
        pltpu.sync_copy(z_hbm.at[pl.ds(s * rpt, rpt)], acc.at[pl.ds(s * rpt, rpt)])
        pltpu.sync_copy(sd_hbm.at[wid], idx_v)
        plsc.subcore_barrier()

        def gather(i, rows, sem):
            pltpu.make_async_copy(yl_hbm.at[idx_v.at[i, 0]], rows, sem).start()

        def gwait(rows, sem):
            pltpu.make_async_copy(yl_hbm, rows, sem).wait()

        def scatter(i, rows, sem):
            return pltpu.async_copy(rows, acc.at[idx_v.at[i, 1]], sem, add=True)

        gather(0, rows_a, sem_ga)

        def pair(p, carry):
            i = 2 * p
            gwait(rows_a, sem_ga)
            gather(i + 1, rows_b, sem_gb)
            da = scatter(i, rows_a, sem_sa)
            gwait(rows_b, sem_gb)
            da.wait()

            @pl.when(i + 2 < steps)
            def _():
                gather(i + 2, rows_a, sem_ga)

            db = scatter(i + 1, rows_b, sem_sb)
            db.wait()
            return carry

        lax.fori_loop(0, steps // 2, pair, 0)
        if steps % 2 == 1:
            gwait(rows_a, sem_ga)
            scatter(steps - 1, rows_a, sem_sa).wait()
        plsc.subcore_barrier()
        pltpu.sync_copy(acc.at[pl.ds(s * rpt, rpt)],
                        out_hbm.at[c, pl.ds(s * rpt, rpt)])

    sd = jnp.stack([src.reshape(NW, steps, CHUNK),
                    dst.reshape(NW, steps, CHUNK)], axis=2)
    return segsum(ylaug, sd, zeros)


def _pre_body(x_ref, wla_ref, wr_ref, b_ref, e_ref, ylaug_ref, yr_ref):
    x = x_ref[...]
    ylaug_ref[...] = (
        jnp.dot(x, wla_ref[...], preferred_element_type=jnp.float32,
                precision=_HIGH) + e_ref[...])
    yr_ref[...] = (
        jnp.dot(x, wr_ref[...], preferred_element_type=jnp.float32,
                precision=_HIGH) + b_ref[...])


def _bn(h0, g_ref, be_ref):
    mu = jnp.mean(h0, axis=0, keepdims=True)
    var = jnp.mean((h0 - mu) * (h0 - mu), axis=0, keepdims=True)
    return (h0 - mu) * lax.rsqrt(var + 1e-5) * g_ref[...] + be_ref[...]


def _bnact_body(n, d, relu, parts_ref, yr_ref, g_ref, be_ref, out_ref):
    agg = parts_ref[0, :n] + parts_ref[1, :n]
    deg = jnp.maximum(agg[:, d:d + 1], 1.0)
    h0 = agg[:, :d] / deg + yr_ref[...]
    h = _bn(h0, g_ref, be_ref)
    out_ref[...] = jnp.maximum(h, 0.0) if relu else h


def kernel(x, edge_index, Wl1, Wr1, b1, g1, be1, Wl2, Wr2, b2, g2, be2):
    n, d = x.shape
    dp = d + 16
    f32 = jnp.float32
    src = edge_index[0]
    dst = edge_index[1]
    ehot = jnp.zeros((1, dp), f32).at[0, d].set(1.0)
    wla1 = jnp.pad(Wl1, ((0, 0), (0, dp - d)))
    wla2 = jnp.pad(Wl2, ((0, 0), (0, dp - d)))
    n_pad = -(-n // (NS * 8)) * (NS * 8)
    zeros = jnp.zeros((n_pad, dp), f32)

    ylaug1, yr1 = pl.pallas_call(
        _pre_body,
        out_shape=[jax.ShapeDtypeStruct((n, dp), f32),
                   jax.ShapeDtypeStruct((n, d), f32)],
    )(x, wla1, Wr1, b1.reshape(1, d), ehot)

    parts1 = _segment_sum_sc(ylaug1, src, dst, zeros)

    h = pl.pallas_call(
        functools.partial(_bnact_body, n, d, True),
        out_shape=jax.ShapeDtypeStruct((n, d), f32),
    )(parts1, yr1, g1.reshape(1, d), be1.reshape(1, d))

    ylaug2, yr2 = pl.pallas_call(
        _pre_body,
        out_shape=[jax.ShapeDtypeStruct((n, dp), f32),
                   jax.ShapeDtypeStruct((n, d), f32)],
    )(h, wla2, Wr2, b2.reshape(1, d), ehot)

    parts2 = _segment_sum_sc(ylaug2, src, dst, zeros)

    out = pl.pallas_call(
        functools.partial(_bnact_body, n, d, False),
        out_shape=jax.ShapeDtypeStruct((n, d), f32),
    )(parts2, yr2, g2.reshape(1, d), be2.reshape(1, d))
    return out


# speedup vs baseline: 1.2953x; 1.2953x over previous
import functools

import jax
import jax.numpy as jnp
from jax import lax
from jax.experimental import pallas as pl
from jax.experimental.pallas import tpu as pltpu
from jax.experimental.pallas import tpu_sc as plsc

NC = 2
NS = 16
NW = NC * NS
CHUNK = 125

_HIGH = lax.Precision.HIGHEST


def _segment_sum_sc(ylaug, src, dst, zeros):
    _, dp = ylaug.shape
    n = zeros.shape[0]
    e = src.shape[0]
    epw = e // NW
    steps = epw // CHUNK
    rpt = n // NS
    mesh = plsc.VectorSubcoreMesh(
        core_axis_name="c", subcore_axis_name="s", num_cores=NC, num_subcores=NS)

    @functools.partial(
        pl.kernel,
        out_type=jax.ShapeDtypeStruct((NC, n, dp), jnp.float32),
        mesh=mesh,
        scratch_types=[
            pltpu.VMEM((2, CHUNK), jnp.int32),
            pltpu.VMEM((2, CHUNK), jnp.int32),
            pltpu.VMEM((CHUNK, dp), jnp.float32),
            pltpu.VMEM((CHUNK, dp), jnp.float32),
            pltpu.VMEM_SHARED((n, dp), jnp.float32),
            pltpu.SemaphoreType.DMA,
            pltpu.SemaphoreType.DMA,
            pltpu.SemaphoreType.DMA,
            pltpu.SemaphoreType.DMA,
        ],
        compiler_params=pltpu.CompilerParams(use_tc_tiling_on_sc=False),
    )
    def segsum(yl_hbm, sd_hbm, z_hbm, out_hbm,
               idx_a, idx_b, rows_a, rows_b, acc, sem_ga, sem_gb, sem_sa, sem_sb):
        c = lax.axis_index("c")
        s = lax.axis_index("s")
        wid = s * NC + c
        pltpu.sync_copy(z_hbm.at[pl.ds(s * rpt, rpt)], acc.at[pl.ds(s * rpt, rpt)])
        plsc.subcore_barrier()

        def load_idx(i, idx):
            pltpu.sync_copy(sd_hbm.at[wid, i], idx)

        def gather(idx, rows, sem):
            pltpu.make_async_copy(yl_hbm.at[idx.at[0]], rows, sem).start()

        def gwait(rows, sem):
            pltpu.make_async_copy(yl_hbm, rows, sem).wait()

        def scatter(idx, rows, sem):
            return pltpu.async_copy(rows, acc.at[idx.at[1]], sem, add=True)

        load_idx(0, idx_a)
        gather(idx_a, rows_a, sem_ga)
        load_idx(1, idx_b)

        def pair(p, carry):
            i = 2 * p
            gwait(rows_a, sem_ga)
            gather(idx_b, rows_b, sem_gb)
            da = scatter(idx_a, rows_a, sem_sa)
            gwait(rows_b, sem_gb)
            db = scatter(idx_b, rows_b, sem_sb)
            da.wait()

            @pl.when(i + 2 < steps)
            def _():
                load_idx(i + 2, idx_a)
                gather(idx_a, rows_a, sem_ga)

            db.wait()

            @pl.when(i + 3 < steps)
            def _():
                load_idx(i + 3, idx_b)

            return carry

        lax.fori_loop(0, steps // 2, pair, 0)
        if steps % 2 == 1:
            gwait(rows_a, sem_ga)
            scatter(idx_a, rows_a, sem_sa).wait()
        plsc.subcore_barrier()
        pltpu.sync_copy(acc.at[pl.ds(s * rpt, rpt)],
                        out_hbm.at[c, pl.ds(s * rpt, rpt)])

    sd = jnp.stack([src.reshape(NW, steps, CHUNK),
                    dst.reshape(NW, steps, CHUNK)], axis=2)
    return segsum(ylaug, sd, zeros)


def _pre_body(x_ref, wla_ref, wr_ref, b_ref, e_ref, ylaug_ref, yr_ref):
    x = x_ref[...]
    ylaug_ref[...] = (
        jnp.dot(x, wla_ref[...], preferred_element_type=jnp.float32,
                precision=_HIGH) + e_ref[...])
    yr_ref[...] = (
        jnp.dot(x, wr_ref[...], preferred_element_type=jnp.float32,
                precision=_HIGH) + b_ref[...])


def _bn(h0, g_ref, be_ref):
    mu = jnp.mean(h0, axis=0, keepdims=True)
    var = jnp.mean((h0 - mu) * (h0 - mu), axis=0, keepdims=True)
    return (h0 - mu) * lax.rsqrt(var + 1e-5) * g_ref[...] + be_ref[...]


def _bnact_body(n, d, relu, parts_ref, yr_ref, g_ref, be_ref, out_ref):
    agg = parts_ref[0, :n] + parts_ref[1, :n]
    deg = jnp.maximum(agg[:, d:d + 1], 1.0)
    h0 = agg[:, :d] / deg + yr_ref[...]
    h = _bn(h0, g_ref, be_ref)
    out_ref[...] = jnp.maximum(h, 0.0) if relu else h


def kernel(x, edge_index, Wl1, Wr1, b1, g1, be1, Wl2, Wr2, b2, g2, be2):
    n, d = x.shape
    dp = d + 16
    f32 = jnp.float32
    src = edge_index[0]
    dst = edge_index[1]
    ehot = jnp.zeros((1, dp), f32).at[0, d].set(1.0)
    wla1 = jnp.pad(Wl1, ((0, 0), (0, dp - d)))
    wla2 = jnp.pad(Wl2, ((0, 0), (0, dp - d)))
    n_pad = -(-n // (NS * 8)) * (NS * 8)
    zeros = jnp.zeros((n_pad, dp), f32)

    ylaug1, yr1 = pl.pallas_call(
        _pre_body,
        out_shape=[jax.ShapeDtypeStruct((n, dp), f32),
                   jax.ShapeDtypeStruct((n, d), f32)],
    )(x, wla1, Wr1, b1.reshape(1, d), ehot)

    parts1 = _segment_sum_sc(ylaug1, src, dst, zeros)

    h = pl.pallas_call(
        functools.partial(_bnact_body, n, d, True),
        out_shape=jax.ShapeDtypeStruct((n, d), f32),
    )(parts1, yr1, g1.reshape(1, d), be1.reshape(1, d))

    ylaug2, yr2 = pl.pallas_call(
        _pre_body,
        out_shape=[jax.ShapeDtypeStruct((n, dp), f32),
                   jax.ShapeDtypeStruct((n, d), f32)],
    )(h, wla2, Wr2, b2.reshape(1, d), ehot)

    parts2 = _segment_sum_sc(ylaug2, src, dst, zeros)

    out = pl.pallas_call(
        functools.partial(_bnact_body, n, d, False),
        out_shape=jax.ShapeDtypeStruct((n, d), f32),
    )(parts2, yr2, g2.reshape(1, d), be2.reshape(1, d))
    return out


# speedup vs baseline: 1.3545x; 1.0457x over previous
import functools

import jax
import jax.numpy as jnp
from jax import lax
from jax.experimental import pallas as pl
from jax.experimental.pallas import tpu as pltpu
from jax.experimental.pallas import tpu_sc as plsc

NC = 2
NS = 16
NW = NC * NS
CHUNK = 125

_HIGH = lax.Precision.DEFAULT


def _segment_sum_sc(ylaug, src, dst, zeros):
    _, dp = ylaug.shape
    n = zeros.shape[0]
    e = src.shape[0]
    epw = e // NW
    steps = epw // CHUNK
    rpt = n // NS
    mesh = plsc.VectorSubcoreMesh(
        core_axis_name="c", subcore_axis_name="s", num_cores=NC, num_subcores=NS)

    @functools.partial(
        pl.kernel,
        out_type=jax.ShapeDtypeStruct((NC, n, dp), jnp.float32),
        mesh=mesh,
        scratch_types=[
            pltpu.VMEM((2, CHUNK), jnp.int32),
            pltpu.VMEM((2, CHUNK), jnp.int32),
            pltpu.VMEM((CHUNK, dp), jnp.float32),
            pltpu.VMEM((CHUNK, dp), jnp.float32),
            pltpu.VMEM_SHARED((n, dp), jnp.float32),
            pltpu.SemaphoreType.DMA,
            pltpu.SemaphoreType.DMA,
            pltpu.SemaphoreType.DMA,
            pltpu.SemaphoreType.DMA,
        ],
        compiler_params=pltpu.CompilerParams(use_tc_tiling_on_sc=False),
    )
    def segsum(yl_hbm, sd_hbm, z_hbm, out_hbm,
               idx_a, idx_b, rows_a, rows_b, acc, sem_ga, sem_gb, sem_sa, sem_sb):
        c = lax.axis_index("c")
        s = lax.axis_index("s")
        wid = s * NC + c
        pltpu.sync_copy(z_hbm.at[pl.ds(s * rpt, rpt)], acc.at[pl.ds(s * rpt, rpt)])
        plsc.subcore_barrier()

        def load_idx(i, idx):
            pltpu.sync_copy(sd_hbm.at[wid, i], idx)

        def gather(idx, rows, sem):
            pltpu.make_async_copy(yl_hbm.at[idx.at[0]], rows, sem).start()

        def gwait(rows, sem):
            pltpu.make_async_copy(yl_hbm, rows, sem).wait()

        def scatter(idx, rows, sem):
            return pltpu.async_copy(rows, acc.at[idx.at[1]], sem, add=True)

        load_idx(0, idx_a)
        gather(idx_a, rows_a, sem_ga)
        load_idx(1, idx_b)

        def pair(p, carry):
            i = 2 * p
            gwait(rows_a, sem_ga)
            gather(idx_b, rows_b, sem_gb)
            da = scatter(idx_a, rows_a, sem_sa)
            gwait(rows_b, sem_gb)
            db = scatter(idx_b, rows_b, sem_sb)
            da.wait()

            @pl.when(i + 2 < steps)
            def _():
                load_idx(i + 2, idx_a)
                gather(idx_a, rows_a, sem_ga)

            db.wait()

            @pl.when(i + 3 < steps)
            def _():
                load_idx(i + 3, idx_b)

            return carry

        lax.fori_loop(0, steps // 2, pair, 0)
        if steps % 2 == 1:
            gwait(rows_a, sem_ga)
            scatter(idx_a, rows_a, sem_sa).wait()
        plsc.subcore_barrier()
        pltpu.sync_copy(acc.at[pl.ds(s * rpt, rpt)],
                        out_hbm.at[c, pl.ds(s * rpt, rpt)])

    sd = jnp.stack([src.reshape(NW, steps, CHUNK),
                    dst.reshape(NW, steps, CHUNK)], axis=2)
    return segsum(ylaug, sd, zeros)


def _pre_body(x_ref, wla_ref, wr_ref, b_ref, e_ref, ylaug_ref, yr_ref):
    x = x_ref[...]
    ylaug_ref[...] = (
        jnp.dot(x, wla_ref[...], preferred_element_type=jnp.float32,
                precision=_HIGH) + e_ref[...])
    yr_ref[...] = (
        jnp.dot(x, wr_ref[...], preferred_element_type=jnp.float32,
                precision=_HIGH) + b_ref[...])


def _bn(h0, g_ref, be_ref):
    mu = jnp.mean(h0, axis=0, keepdims=True)
    var = jnp.mean((h0 - mu) * (h0 - mu), axis=0, keepdims=True)
    return (h0 - mu) * lax.rsqrt(var + 1e-5) * g_ref[...] + be_ref[...]


def _bnact_body(n, d, relu, parts_ref, yr_ref, g_ref, be_ref, out_ref):
    agg = parts_ref[0, :n] + parts_ref[1, :n]
    deg = jnp.maximum(agg[:, d:d + 1], 1.0)
    h0 = agg[:, :d] / deg + yr_ref[...]
    h = _bn(h0, g_ref, be_ref)
    out_ref[...] = jnp.maximum(h, 0.0) if relu else h


def kernel(x, edge_index, Wl1, Wr1, b1, g1, be1, Wl2, Wr2, b2, g2, be2):
    n, d = x.shape
    dp = d + 16
    f32 = jnp.float32
    src = edge_index[0]
    dst = edge_index[1]
    ehot = jnp.zeros((1, dp), f32).at[0, d].set(1.0)
    wla1 = jnp.pad(Wl1, ((0, 0), (0, dp - d)))
    wla2 = jnp.pad(Wl2, ((0, 0), (0, dp - d)))
    n_pad = -(-n // (NS * 8)) * (NS * 8)
    zeros = jnp.zeros((n_pad, dp), f32)

    ylaug1, yr1 = pl.pallas_call(
        _pre_body,
        out_shape=[jax.ShapeDtypeStruct((n, dp), f32),
                   jax.ShapeDtypeStruct((n, d), f32)],
    )(x, wla1, Wr1, b1.reshape(1, d), ehot)

    parts1 = _segment_sum_sc(ylaug1, src, dst, zeros)

    h = pl.pallas_call(
        functools.partial(_bnact_body, n, d, True),
        out_shape=jax.ShapeDtypeStruct((n, d), f32),
    )(parts1, yr1, g1.reshape(1, d), be1.reshape(1, d))

    ylaug2, yr2 = pl.pallas_call(
        _pre_body,
        out_shape=[jax.ShapeDtypeStruct((n, dp), f32),
                   jax.ShapeDtypeStruct((n, d), f32)],
    )(h, wla2, Wr2, b2.reshape(1, d), ehot)

    parts2 = _segment_sum_sc(ylaug2, src, dst, zeros)

    out = pl.pallas_call(
        functools.partial(_bnact_body, n, d, False),
        out_shape=jax.ShapeDtypeStruct((n, d), f32),
    )(parts2, yr2, g2.reshape(1, d), be2.reshape(1, d))
    return out


# speedup vs baseline: 1.6173x; 1.1940x over previous
import functools

import jax
import jax.numpy as jnp
from jax import lax
from jax.experimental import pallas as pl
from jax.experimental.pallas import tpu as pltpu
from jax.experimental.pallas import tpu_sc as plsc

NC = 2
NS = 16
NW = NC * NS
CHUNK = 125

_PREC = lax.Precision.DEFAULT


def _segment_sum_sc(yl, sd, zeros, z16, ones16, with_deg):
    _, d = yl.shape
    n = zeros.shape[0]
    steps = sd.shape[1]
    rpt = n // NS
    mesh = plsc.VectorSubcoreMesh(
        core_axis_name="c", subcore_axis_name="s", num_cores=NC, num_subcores=NS)

    out_type = [jax.ShapeDtypeStruct((NC, n, d), jnp.float32)]
    scratch = [
        pltpu.VMEM((2, CHUNK), jnp.int32),
        pltpu.VMEM((2, CHUNK), jnp.int32),
        pltpu.VMEM((CHUNK, d), jnp.float32),
        pltpu.VMEM((CHUNK, d), jnp.float32),
        pltpu.VMEM_SHARED((n, d), jnp.float32),
        pltpu.SemaphoreType.DMA,
        pltpu.SemaphoreType.DMA,
        pltpu.SemaphoreType.DMA,
        pltpu.SemaphoreType.DMA,
    ]
    if with_deg:
        out_type.append(jax.ShapeDtypeStruct((NC, n, 16), jnp.float32))
        scratch += [
            pltpu.VMEM((CHUNK, 16), jnp.float32),
            pltpu.VMEM_SHARED((n, 16), jnp.float32),
            pltpu.SemaphoreType.DMA,
        ]

    @functools.partial(
        pl.kernel, out_type=out_type, mesh=mesh, scratch_types=scratch,
        compiler_params=pltpu.CompilerParams(use_tc_tiling_on_sc=False),
    )
    def segsum(yl_hbm, sd_hbm, z_hbm, z16_hbm, o16_hbm, *rest):
        if with_deg:
            (out_hbm, deg_hbm, idx_a, idx_b, rows_a, rows_b, acc,
             sem_ga, sem_gb, sem_sa, sem_sb, ones_v, acc16, sem_d) = rest
        else:
            (out_hbm, idx_a, idx_b, rows_a, rows_b, acc,
             sem_ga, sem_gb, sem_sa, sem_sb) = rest
        c = lax.axis_index("c")
        s = lax.axis_index("s")
        wid = s * NC + c
        pltpu.sync_copy(z_hbm.at[pl.ds(s * rpt, rpt)], acc.at[pl.ds(s * rpt, rpt)])
        if with_deg:
            pltpu.sync_copy(z16_hbm.at[pl.ds(s * rpt, rpt)],
                            acc16.at[pl.ds(s * rpt, rpt)])
            pltpu.sync_copy(o16_hbm, ones_v)
        plsc.subcore_barrier()

        def load_idx(i, idx):
            pltpu.sync_copy(sd_hbm.at[wid, i], idx)

        def gather(idx, rows, sem):
            pltpu.make_async_copy(yl_hbm.at[idx.at[0]], rows, sem).start()

        def gwait(rows, sem):
            pltpu.make_async_copy(yl_hbm, rows, sem).wait()

        def scatter(idx, rows, sem):
            return pltpu.async_copy(rows, acc.at[idx.at[1]], sem, add=True)

        def deg_scatter(idx):
            if with_deg:
                return pltpu.async_copy(ones_v, acc16.at[idx.at[1]], sem_d,
                                        add=True)
            return None

        load_idx(0, idx_a)
        gather(idx_a, rows_a, sem_ga)
        load_idx(1, idx_b)

        def pair(p, carry):
            i = 2 * p
            gwait(rows_a, sem_ga)
            gather(idx_b, rows_b, sem_gb)
            da = scatter(idx_a, rows_a, sem_sa)
            dd = deg_scatter(idx_a)
            gwait(rows_b, sem_gb)
            db = scatter(idx_b, rows_b, sem_sb)
            da.wait()
            if dd is not None:
                dd.wait()

            @pl.when(i + 2 < steps)
            def _():
                load_idx(i + 2, idx_a)
                gather(idx_a, rows_a, sem_ga)

            db.wait()
            de = deg_scatter(idx_b)
            if de is not None:
                de.wait()

            @pl.when(i + 3 < steps)
            def _():
                load_idx(i + 3, idx_b)

            return carry

        lax.fori_loop(0, steps // 2, pair, 0)
        if steps % 2 == 1:
            gwait(rows_a, sem_ga)
            scatter(idx_a, rows_a, sem_sa).wait()
            dd = deg_scatter(idx_a)
            if dd is not None:
                dd.wait()
        plsc.subcore_barrier()
        pltpu.sync_copy(acc.at[pl.ds(s * rpt, rpt)],
                        out_hbm.at[c, pl.ds(s * rpt, rpt)])
        if with_deg:
            pltpu.sync_copy(acc16.at[pl.ds(s * rpt, rpt)],
                            deg_hbm.at[c, pl.ds(s * rpt, rpt)])

    return segsum(yl, sd, zeros, z16, ones16)


def _pre_body(x_ref, wl_ref, wr_ref, b_ref, yl_ref, yr_ref):
    x = x_ref[...]
    yl_ref[...] = jnp.dot(x, wl_ref[...], preferred_element_type=jnp.float32,
                          precision=_PREC)
    yr_ref[...] = jnp.dot(x, wr_ref[...], preferred_element_type=jnp.float32,
                          precision=_PREC) + b_ref[...]


def _bn(h0, g_ref, be_ref):
    mu = jnp.mean(h0, axis=0, keepdims=True)
    var = jnp.mean((h0 - mu) * (h0 - mu), axis=0, keepdims=True)
    return (h0 - mu) * lax.rsqrt(var + 1e-5) * g_ref[...] + be_ref[...]


def _bnact_body(n, relu, parts_ref, degp_ref, yr_ref, g_ref, be_ref, out_ref):
    agg = parts_ref[0, :n] + parts_ref[1, :n]
    deg = jnp.maximum(degp_ref[0, :n, 0:1] + degp_ref[1, :n, 0:1], 1.0)
    h0 = agg / deg + yr_ref[...]
    h = _bn(h0, g_ref, be_ref)
    out_ref[...] = jnp.maximum(h, 0.0) if relu else h


def kernel(x, edge_index, Wl1, Wr1, b1, g1, be1, Wl2, Wr2, b2, g2, be2):
    n, d = x.shape
    f32 = jnp.float32
    e = edge_index.shape[1]
    epw = e // NW
    steps = epw // CHUNK
    sd = jnp.stack([edge_index[0].reshape(NW, steps, CHUNK),
                    edge_index[1].reshape(NW, steps, CHUNK)], axis=2)
    n_pad = -(-n // (NS * 8)) * (NS * 8)
    zeros = jnp.zeros((n_pad, d), f32)
    z16 = jnp.zeros((n_pad, 16), f32)
    ones16 = jnp.zeros((CHUNK, 16), f32).at[:, 0].set(1.0)

    yl1, yr1 = pl.pallas_call(
        _pre_body,
        out_shape=[jax.ShapeDtypeStruct((n, d), f32),
                   jax.ShapeDtypeStruct((n, d), f32)],
    )(x, Wl1, Wr1, b1.reshape(1, d))

    parts1, degp = _segment_sum_sc(yl1, sd, zeros, z16, ones16, True)

    h = pl.pallas_call(
        functools.partial(_bnact_body, n, True),
        out_shape=jax.ShapeDtypeStruct((n, d), f32),
    )(parts1, degp, yr1, g1.reshape(1, d), be1.reshape(1, d))

    yl2, yr2 = pl.pallas_call(
        _pre_body,
        out_shape=[jax.ShapeDtypeStruct((n, d), f32),
                   jax.ShapeDtypeStruct((n, d), f32)],
    )(h, Wl2, Wr2, b2.reshape(1, d))

    parts2 = _segment_sum_sc(yl2, sd, zeros, z16, ones16, False)[0]

    out = pl.pallas_call(
        functools.partial(_bnact_body, n, False),
        out_shape=jax.ShapeDtypeStruct((n, d), f32),
    )(parts2, degp, yr2, g2.reshape(1, d), be2.reshape(1, d))
    return out
